# baseline (speedup 1.0000x reference)
import functools

import jax
import jax.numpy as jnp
from jax import lax
from jax.experimental import pallas as pl
from jax.experimental.pallas import tpu as pltpu
from jax.experimental.pallas import tpu_sc as plsc

_H = 16
_S = 2048
_NB = 15
_RB = _S // 8
_W2 = 4112

_mesh = plsc.VectorSubcoreMesh(
    core_axis_name="c", subcore_axis_name="s", num_cores=2, num_subcores=16
)


@functools.partial(
    pl.kernel,
    out_type=jax.ShapeDtypeStruct((_H, _S, _S), jnp.float32),
    mesh=_mesh,
    scratch_types=[
        pltpu.VMEM((_NB * _NB * _H,), jnp.float32),
        pltpu.VMEM((16, _W2), jnp.float32),
        pltpu.SemaphoreType.DMA,
    ],
    compiler_params=pltpu.CompilerParams(
        needs_layout_passes=False, use_tc_tiling_on_sc=False
    ),
)
def _rpb_sc(table_hbm, out_hbm, table_v, strip_v, sem):
    h = lax.axis_index("s")
    half = lax.axis_index("c")

    pltpu.sync_copy(table_hbm, table_v)

    iota = lax.iota(jnp.int32, 16)
    fj = iota & 7
    ksub = iota >> 3

    def tab_idx(fi, fjv, idx1):
        return (fi - fjv + 7) * (_NB * _H) + idx1 * _H + h

    pat_l = [plsc.load_gather(table_v, [tab_idx(fi, fj, 14)]) for fi in range(8)]
    pat_r = [plsc.load_gather(table_v, [tab_idx(fi, fj, 0)]) for fi in range(8)]

    for fi in range(8):
        for u in range(7):
            idx1 = 14 - (2 * u + ksub)
            vals = plsc.load_gather(table_v, [tab_idx(fi, fj, idx1)])
            strip_v[fi, pl.ds(1992 + 16 * u, 16)] = vals
            strip_v[8 + fi, pl.ds(2000 + 16 * u, 16)] = vals

    def fill_body(t, carry):
        c = 16 * t
        for fi in range(8):
            strip_v[fi, pl.ds(8 + c, 16)] = pat_l[fi]
            strip_v[8 + fi, pl.ds(16 + c, 16)] = pat_l[fi]
            strip_v[fi, pl.ds(2104 + c, 16)] = pat_r[fi]
            strip_v[8 + fi, pl.ds(2112 + c, 16)] = pat_r[fi]
        return carry

    lax.fori_loop(0, 124, fill_body, 0)
    for fi in range(8):
        strip_v[fi, pl.ds(4080, 16)] = pat_r[fi]
        strip_v[8 + fi, pl.ds(4096, 16)] = pat_r[fi]

    r0 = half * (_RB // 2)

    def pair_copy(t):
        ri = r0 + 2 * t
        c0 = 8 * (_RB - ri)
        return pltpu.make_async_copy(
            strip_v.at[:, pl.ds(c0, _S)],
            out_hbm.at[h, pl.ds(8 * ri, 16), :],
            sem,
        )

    def write_body(t, carry):
        pair_copy(t).start()

        @pl.when(t >= 3)
        def _():
            pair_copy(0).wait()

        return carry

    lax.fori_loop(0, _RB // 4, write_body, 0)
    for _ in range(3):
        pair_copy(0).wait()


def kernel(seq_len, table):
    del seq_len
    out = _rpb_sc(table.reshape(-1))
    return out[None]
